# Initial kernel scaffold; baseline (speedup 1.0000x reference)
import jax
import jax.numpy as jnp
from jax.experimental import pallas as pl


def kernel(student_emb, teacher_emb, W1, b1, W2, b2, W3, b3):
    raise NotImplementedError("write your pallas kernel here")


# speedup vs baseline: 1.2697x; 1.2697x over previous
import jax
import jax.numpy as jnp
from jax.experimental import pallas as pl

_BLOCK = 1024


def _router_block(s_ref, t_ref, w1_ref, b1_ref, w2_ref, b2_ref, w3_ref,
                  b3_ref, out_ref, logit_ref):
    s = s_ref[...]
    h1 = jnp.maximum(
        jnp.dot(s, w1_ref[...], preferred_element_type=jnp.float32)
        + b1_ref[...], 0.0)
    h2 = jnp.maximum(
        jnp.dot(h1, w2_ref[...], preferred_element_type=jnp.float32)
        + b2_ref[...], 0.0)
    logit = jnp.dot(h2, w3_ref[...], preferred_element_type=jnp.float32) \
        + b3_ref[...]
    out_ref[...] = jnp.where(logit > 0.0, t_ref[...], s)
    logit_ref[...] = logit


def kernel(student_emb, teacher_emb, W1, b1, W2, b2, W3, b3):
    batch, dim = student_emb.shape
    hidden = W1.shape[1]
    half = W2.shape[1]
    grid = (batch // _BLOCK,)

    out, logits = pl.pallas_call(
        _router_block,
        grid=grid,
        in_specs=[
            pl.BlockSpec((_BLOCK, dim), lambda i: (i, 0)),
            pl.BlockSpec((_BLOCK, dim), lambda i: (i, 0)),
            pl.BlockSpec((dim, hidden), lambda i: (0, 0)),
            pl.BlockSpec((1, hidden), lambda i: (0, 0)),
            pl.BlockSpec((hidden, half), lambda i: (0, 0)),
            pl.BlockSpec((1, half), lambda i: (0, 0)),
            pl.BlockSpec((half, 1), lambda i: (0, 0)),
            pl.BlockSpec((1, 1), lambda i: (0, 0)),
        ],
        out_specs=[
            pl.BlockSpec((_BLOCK, dim), lambda i: (i, 0)),
            pl.BlockSpec((_BLOCK, 1), lambda i: (i, 0)),
        ],
        out_shape=[
            jax.ShapeDtypeStruct((batch, dim), jnp.float32),
            jax.ShapeDtypeStruct((batch, 1), jnp.float32),
        ],
    )(student_emb, teacher_emb, W1, b1.reshape(1, hidden), W2,
      b2.reshape(1, half), W3, b3.reshape(1, 1))

    use_teacher = logits[:, 0] > 0.0
    return (out, use_teacher)


# speedup vs baseline: 1.3340x; 1.0507x over previous
import jax
import jax.numpy as jnp
from jax.experimental import pallas as pl

_BLOCK = 2048


def _router_block(s_ref, t_ref, w1_ref, b1_ref, w2_ref, b2_ref, w3_ref,
                  b3_ref, out_ref, logit_ref):
    s = s_ref[...]
    h1 = jnp.maximum(
        jnp.dot(s, w1_ref[...], preferred_element_type=jnp.float32)
        + b1_ref[...], 0.0)
    h2 = jnp.maximum(
        jnp.dot(h1, w2_ref[...], preferred_element_type=jnp.float32)
        + b2_ref[...], 0.0)
    logit = jnp.dot(h2, w3_ref[...], preferred_element_type=jnp.float32) \
        + b3_ref[...]
    out_ref[...] = jnp.where(logit > 0.0, t_ref[...], s)
    logit_ref[...] = logit


def kernel(student_emb, teacher_emb, W1, b1, W2, b2, W3, b3):
    batch, dim = student_emb.shape
    hidden = W1.shape[1]
    half = W2.shape[1]
    grid = (batch // _BLOCK,)

    out, logits = pl.pallas_call(
        _router_block,
        grid=grid,
        in_specs=[
            pl.BlockSpec((_BLOCK, dim), lambda i: (i, 0)),
            pl.BlockSpec((_BLOCK, dim), lambda i: (i, 0)),
            pl.BlockSpec((dim, hidden), lambda i: (0, 0)),
            pl.BlockSpec((1, hidden), lambda i: (0, 0)),
            pl.BlockSpec((hidden, half), lambda i: (0, 0)),
            pl.BlockSpec((1, half), lambda i: (0, 0)),
            pl.BlockSpec((half, 1), lambda i: (0, 0)),
            pl.BlockSpec((1, 1), lambda i: (0, 0)),
        ],
        out_specs=[
            pl.BlockSpec((_BLOCK, dim), lambda i: (i, 0)),
            pl.BlockSpec((_BLOCK, 1), lambda i: (i, 0)),
        ],
        out_shape=[
            jax.ShapeDtypeStruct((batch, dim), jnp.float32),
            jax.ShapeDtypeStruct((batch, 1), jnp.float32),
        ],
    )(student_emb, teacher_emb, W1, b1.reshape(1, hidden), W2,
      b2.reshape(1, half), W3, b3.reshape(1, 1))

    use_teacher = logits[:, 0] > 0.0
    return (out, use_teacher)
